# baseline (speedup 1.0000x reference)
import functools

import jax
import jax.numpy as jnp
from jax import lax
from jax.experimental import pallas as pl
from jax.experimental.pallas import tpu as pltpu
from jax.experimental.pallas import tpu_sc as plsc

NC = 2
NS = 16
NW = NC * NS
L = 16
CHUNK = 2048


def _interp_kernel(nf, d, h, w, n, coords_hbm, vols_hbm, out_hbm,
                   coord_refs, cidx_refs, idx_refs, corner_refs, w_refs,
                   out_v, gsem0, gsem1, csem0, csem1):
    b_per_w = n // NW
    n_chunks = b_per_w // CHUNK
    n_pairs = n_chunks // 2
    wid = lax.axis_index("s") * NC + lax.axis_index("c")
    base = wid * b_per_w
    last_cbase = base + (n_chunks - 1) * CHUNK

    hw = h * w
    dhw = d * hw
    gsems = [gsem0, gsem1]
    csems = [csem0, csem1]
    lanes4 = lax.iota(jnp.int32, 16) * 4

    def fire_coords(ci, p):
        cbase = lax.min(base + ci * CHUNK, last_cbase)
        czi, cyi, cxi, cti = cidx_refs[p]

        def cidx_body(i, _):
            sl = pl.ds(i * L, L)
            v = lanes4 + (cbase * 4 + i * (L * 4))
            czi[sl] = v
            cyi[sl] = v + 1
            cxi[sl] = v + 2
            cti[sl] = v + 3
            return 0

        lax.fori_loop(0, CHUNK // L, cidx_body, 0)
        for cref, dst in zip(cidx_refs[p], coord_refs[p]):
            pltpu.async_copy(coords_hbm.at[cref], dst, csems[p])

    def compute_and_fire(ci, p):
        for cref, dst in zip(cidx_refs[p], coord_refs[p]):
            pltpu.make_async_copy(coords_hbm.at[cref], dst, csems[p]).wait()

        zc_v, yc_v, xc_v, tc_v = coord_refs[p]
        iref = idx_refs[p]
        wzr, wyr, wxr = w_refs[p]

        def idx_body(i, _):
            sl = pl.ds(i * L, L)
            zc = zc_v[sl]
            yc = yc_v[sl]
            xc = xc_v[sl]
            tc = tc_v[sl]
            sz = zc * float(d - 1)
            sy = yc * float(h - 1)
            sx = xc * float(w - 1)
            iz = sz.astype(jnp.int32)
            iy = sy.astype(jnp.int32)
            ix = sx.astype(jnp.int32)
            wzr[sl] = sz - iz.astype(jnp.float32)
            wyr[sl] = sy - iy.astype(jnp.float32)
            wxr[sl] = sx - ix.astype(jnp.float32)
            z0 = jnp.clip(iz, 0, d - 1)
            y0 = jnp.clip(iy, 0, h - 1)
            x0 = jnp.clip(ix, 0, w - 1)
            z1 = jnp.minimum(z0 + 1, d - 1)
            y1 = jnp.minimum(y0 + 1, h - 1)
            x1 = jnp.minimum(x0 + 1, w - 1)
            d0 = jnp.abs(tc + 1.0)
            d1 = jnp.abs(tc)
            d2 = jnp.abs(tc - 1.0)
            fi = jnp.where(d1 < d0,
                           jnp.where(d2 < d1, 2, 1),
                           jnp.where(d2 < d0, 2, 0)).astype(jnp.int32)
            b00 = fi * dhw + z0 * hw + y0 * w
            b01 = b00 + (y1 - y0) * w
            b10 = b00 + (z1 - z0) * hw
            b11 = b10 + (y1 - y0) * w
            iref[0][sl] = b00 + x0
            iref[1][sl] = b00 + x1
            iref[2][sl] = b01 + x0
            iref[3][sl] = b01 + x1
            iref[4][sl] = b10 + x0
            iref[5][sl] = b10 + x1
            iref[6][sl] = b11 + x0
            iref[7][sl] = b11 + x1
            return 0

        lax.fori_loop(0, CHUNK // L, idx_body, 0)
        fire_coords(ci + 1, 1 - p)
        for k in range(8):
            pltpu.async_copy(vols_hbm.at[iref[k]], corner_refs[p][k],
                             gsems[p])

    def drain_combine_store(ci, p):
        for k in range(8):
            pltpu.make_async_copy(vols_hbm.at[idx_refs[p][k]],
                                  corner_refs[p][k], gsems[p]).wait()
        cref = corner_refs[p]
        wzr, wyr, wxr = w_refs[p]

        def comb_body(i, _):
            sl = pl.ds(i * L, L)
            wx = wxr[sl]
            wy = wyr[sl]
            wz = wzr[sl]
            c00 = cref[0][sl] * (1 - wx) + cref[1][sl] * wx
            c01 = cref[2][sl] * (1 - wx) + cref[3][sl] * wx
            c10 = cref[4][sl] * (1 - wx) + cref[5][sl] * wx
            c11 = cref[6][sl] * (1 - wx) + cref[7][sl] * wx
            c0 = c00 * (1 - wy) + c01 * wy
            c1 = c10 * (1 - wy) + c11 * wy
            out_v[sl] = c0 * (1 - wz) + c1 * wz
            return 0

        lax.fori_loop(0, CHUNK // L, comb_body, 0)
        cbase = base + ci * CHUNK
        pltpu.sync_copy(out_v, out_hbm.at[pl.ds(cbase, CHUNK)])

    fire_coords(0, 0)
    compute_and_fire(0, 0)

    def pair_body(pi, _):
        ci = pi * 2
        compute_and_fire(ci + 1, 1)
        drain_combine_store(ci, 0)
        compute_and_fire(ci + 2, 0)
        drain_combine_store(ci + 1, 1)
        return 0

    lax.fori_loop(0, n_pairs - 1, pair_body, 0)

    ci = (n_pairs - 1) * 2
    compute_and_fire(ci + 1, 1)
    drain_combine_store(ci, 0)
    drain_combine_store(ci + 1, 1)
    for cref, dst in zip(cidx_refs[0], coord_refs[0]):
        pltpu.make_async_copy(coords_hbm.at[cref], dst, csems[0]).wait()


def kernel(coords, vols):
    n = coords.shape[0]
    nf, d, h, w = vols.shape
    coords_flat = coords.reshape(-1)
    vols_flat = vols.reshape(-1)

    mesh = plsc.VectorSubcoreMesh(core_axis_name="c", subcore_axis_name="s",
                                  num_cores=NC, num_subcores=NS)
    body = functools.partial(_interp_kernel, nf, d, h, w, n)
    run = pl.kernel(
        body,
        out_type=jax.ShapeDtypeStruct((n,), jnp.float32),
        mesh=mesh,
        scratch_types=[
            [[pltpu.VMEM((CHUNK,), jnp.float32) for _ in range(4)]
             for _ in range(2)],
            [[pltpu.VMEM((CHUNK,), jnp.int32) for _ in range(4)]
             for _ in range(2)],
            [[pltpu.VMEM((CHUNK,), jnp.int32) for _ in range(8)]
             for _ in range(2)],
            [[pltpu.VMEM((CHUNK,), jnp.float32) for _ in range(8)]
             for _ in range(2)],
            [[pltpu.VMEM((CHUNK,), jnp.float32) for _ in range(3)]
             for _ in range(2)],
            pltpu.VMEM((CHUNK,), jnp.float32),
            pltpu.SemaphoreType.DMA,
            pltpu.SemaphoreType.DMA,
            pltpu.SemaphoreType.DMA,
            pltpu.SemaphoreType.DMA,
        ],
    )
    out = run(coords_flat, vols_flat)
    return out.reshape(n, 1)


# speedup vs baseline: 5.7432x; 5.7432x over previous
import functools

import jax
import jax.numpy as jnp
from jax import lax
from jax.experimental import pallas as pl
from jax.experimental.pallas import tpu as pltpu
from jax.experimental.pallas import tpu_sc as plsc

NC = 2
NS = 16
NW = NC * NS
L = 16
CHUNK = 2048


def _interp_kernel(nf, d, h, w, n, coords_hbm, vols_hbm, out_hbm,
                   coord_refs, idx_refs, corner_refs, w_refs,
                   out_v, gsem0, gsem1, csem0, csem1):
    b_per_w = n // NW
    n_chunks = b_per_w // CHUNK
    n_pairs = n_chunks // 2
    wid = lax.axis_index("s") * NC + lax.axis_index("c")
    base = wid * b_per_w
    last_cbase = base + (n_chunks - 1) * CHUNK

    hw = h * w
    dhw = d * hw
    gsems = [gsem0, gsem1]
    csems = [csem0, csem1]
    yts = (w // 128) * 1024

    def fire_coords(ci, p):
        cbase = lax.min(base + ci * CHUNK, last_cbase)
        for q, dst in enumerate(coord_refs[p]):
            pltpu.async_copy(coords_hbm.at[pl.ds(q * n + cbase, CHUNK)],
                             dst, csems[p])

    def compute_and_fire(ci, p):
        cbase0 = lax.min(base + ci * CHUNK, last_cbase)
        for q, dst in enumerate(coord_refs[p]):
            pltpu.make_async_copy(
                coords_hbm.at[pl.ds(q * n + cbase0, CHUNK)], dst,
                csems[p]).wait()

        zc_v, yc_v, xc_v, tc_v = coord_refs[p]
        iref = idx_refs[p]
        wzr, wyr, wxr = w_refs[p]

        def idx_body(i, _):
            sl = pl.ds(i * L, L)
            zc = zc_v[sl]
            yc = yc_v[sl]
            xc = xc_v[sl]
            tc = tc_v[sl]
            sz = zc * float(d - 1)
            sy = yc * float(h - 1)
            sx = xc * float(w - 1)
            iz = sz.astype(jnp.int32)
            iy = sy.astype(jnp.int32)
            ix = sx.astype(jnp.int32)
            wzr[sl] = sz - iz.astype(jnp.float32)
            wyr[sl] = sy - iy.astype(jnp.float32)
            wxr[sl] = sx - ix.astype(jnp.float32)
            z0 = jnp.clip(iz, 0, d - 1)
            y0 = jnp.clip(iy, 0, h - 1)
            x0 = jnp.clip(ix, 0, w - 1)
            z1 = jnp.minimum(z0 + 1, d - 1)
            y1 = jnp.minimum(y0 + 1, h - 1)
            x1 = jnp.minimum(x0 + 1, w - 1)
            d0 = jnp.abs(tc + 1.0)
            d1 = jnp.abs(tc)
            d2 = jnp.abs(tc - 1.0)
            fi = jnp.where(d1 < d0,
                           jnp.where(d2 < d1, 2, 1),
                           jnp.where(d2 < d0, 2, 0)).astype(jnp.int32)
            za = (fi * d + z0) * hw
            zb = za + (z1 - z0) * hw
            ya = (y0 >> 3) * yts + ((y0 & 7) << 7)
            yb = (y1 >> 3) * yts + ((y1 & 7) << 7)
            xa = ((x0 >> 7) << 10) + (x0 & 127)
            xb = ((x1 >> 7) << 10) + (x1 & 127)
            b00 = za + ya
            b01 = za + yb
            b10 = zb + ya
            b11 = zb + yb
            iref[0][sl] = b00 + xa
            iref[1][sl] = b00 + xb
            iref[2][sl] = b01 + xa
            iref[3][sl] = b01 + xb
            iref[4][sl] = b10 + xa
            iref[5][sl] = b10 + xb
            iref[6][sl] = b11 + xa
            iref[7][sl] = b11 + xb
            return 0

        lax.fori_loop(0, CHUNK // L, idx_body, 0)
        fire_coords(ci + 1, 1 - p)
        for k in range(8):
            pltpu.async_copy(vols_hbm.at[iref[k]], corner_refs[p][k],
                             gsems[p])

    def drain_combine_store(ci, p):
        for k in range(8):
            pltpu.make_async_copy(vols_hbm.at[idx_refs[p][k]],
                                  corner_refs[p][k], gsems[p]).wait()
        cref = corner_refs[p]
        wzr, wyr, wxr = w_refs[p]

        def comb_body(i, _):
            sl = pl.ds(i * L, L)
            wx = wxr[sl]
            wy = wyr[sl]
            wz = wzr[sl]
            c00 = cref[0][sl] * (1 - wx) + cref[1][sl] * wx
            c01 = cref[2][sl] * (1 - wx) + cref[3][sl] * wx
            c10 = cref[4][sl] * (1 - wx) + cref[5][sl] * wx
            c11 = cref[6][sl] * (1 - wx) + cref[7][sl] * wx
            c0 = c00 * (1 - wy) + c01 * wy
            c1 = c10 * (1 - wy) + c11 * wy
            out_v[sl] = c0 * (1 - wz) + c1 * wz
            return 0

        lax.fori_loop(0, CHUNK // L, comb_body, 0)
        cbase = base + ci * CHUNK
        pltpu.sync_copy(out_v, out_hbm.at[pl.ds(cbase, CHUNK)])

    fire_coords(0, 0)
    compute_and_fire(0, 0)

    def pair_body(pi, _):
        ci = pi * 2
        compute_and_fire(ci + 1, 1)
        drain_combine_store(ci, 0)
        compute_and_fire(ci + 2, 0)
        drain_combine_store(ci + 1, 1)
        return 0

    lax.fori_loop(0, n_pairs - 1, pair_body, 0)

    ci = (n_pairs - 1) * 2
    compute_and_fire(ci + 1, 1)
    drain_combine_store(ci, 0)
    drain_combine_store(ci + 1, 1)
    for q, dst in enumerate(coord_refs[0]):
        pltpu.make_async_copy(
            coords_hbm.at[pl.ds(q * n + last_cbase, CHUNK)], dst,
            csems[0]).wait()


def kernel(coords, vols):
    n = coords.shape[0]
    nf, d, h, w = vols.shape
    coords_flat = coords.T.reshape(-1)
    vols_flat = (vols.reshape(nf, d, h // 8, 8, w // 128, 128)
                 .transpose(0, 1, 2, 4, 3, 5).reshape(-1))

    mesh = plsc.VectorSubcoreMesh(core_axis_name="c", subcore_axis_name="s",
                                  num_cores=NC, num_subcores=NS)
    body = functools.partial(_interp_kernel, nf, d, h, w, n)
    run = pl.kernel(
        body,
        out_type=jax.ShapeDtypeStruct((n,), jnp.float32),
        mesh=mesh,
        scratch_types=[
            [[pltpu.VMEM((CHUNK,), jnp.float32) for _ in range(4)]
             for _ in range(2)],
            [[pltpu.VMEM((CHUNK,), jnp.int32) for _ in range(8)]
             for _ in range(2)],
            [[pltpu.VMEM((CHUNK,), jnp.float32) for _ in range(8)]
             for _ in range(2)],
            [[pltpu.VMEM((CHUNK,), jnp.float32) for _ in range(3)]
             for _ in range(2)],
            pltpu.VMEM((CHUNK,), jnp.float32),
            pltpu.SemaphoreType.DMA,
            pltpu.SemaphoreType.DMA,
            pltpu.SemaphoreType.DMA,
            pltpu.SemaphoreType.DMA,
        ],
    )
    out = run(coords_flat, vols_flat)
    return out.reshape(n, 1)


# speedup vs baseline: 6.0033x; 1.0453x over previous
import functools

import jax
import jax.numpy as jnp
from jax import lax
from jax.experimental import pallas as pl
from jax.experimental.pallas import tpu as pltpu
from jax.experimental.pallas import tpu_sc as plsc

NC = 2
NS = 16
NW = NC * NS
L = 16
CHUNK = 2048


def _interp_kernel(nf, d, h, w, n, coords_hbm, vols_hbm, out_hbm,
                   coord_refs, idx_refs, corner_refs, w_refs,
                   out_v, gsem0, gsem1, csem0, csem1):
    b_per_w = n // NW
    n_chunks = b_per_w // CHUNK
    n_pairs = n_chunks // 2
    wid = lax.axis_index("s") * NC + lax.axis_index("c")
    base = wid * b_per_w
    last_cbase = base + (n_chunks - 1) * CHUNK

    hw = h * w
    dhw = d * hw
    gsems = [gsem0, gsem1]
    csems = [csem0, csem1]
    yts = (w // 128) * 1024

    def fire_coords(ci, p):
        cbase = lax.min(base + ci * CHUNK, last_cbase)
        pltpu.async_copy(coords_hbm.at[pl.ds(cbase * 4, CHUNK * 4)],
                         coord_refs[p], csems[p])

    def compute_and_fire(ci, p):
        cbase0 = base + ci * CHUNK
        pltpu.make_async_copy(coords_hbm.at[pl.ds(cbase0 * 4, CHUNK * 4)],
                              coord_refs[p], csems[p]).wait()

        cbuf = coord_refs[p]
        iref = idx_refs[p]
        wzr, wyr, wxr = w_refs[p]

        def idx_body(i, _):
            sl = pl.ds(i * L, L)
            off = ((i >> 3) << 9) + ((i & 7) << 4)
            zc = cbuf[pl.ds(off, L)]
            yc = cbuf[pl.ds(off + 128, L)]
            xc = cbuf[pl.ds(off + 256, L)]
            tc = cbuf[pl.ds(off + 384, L)]
            sz = zc * float(d - 1)
            sy = yc * float(h - 1)
            sx = xc * float(w - 1)
            iz = sz.astype(jnp.int32)
            iy = sy.astype(jnp.int32)
            ix = sx.astype(jnp.int32)
            wzr[sl] = sz - iz.astype(jnp.float32)
            wyr[sl] = sy - iy.astype(jnp.float32)
            wxr[sl] = sx - ix.astype(jnp.float32)
            z0 = jnp.clip(iz, 0, d - 1)
            y0 = jnp.clip(iy, 0, h - 1)
            x0 = jnp.clip(ix, 0, w - 1)
            z1 = jnp.minimum(z0 + 1, d - 1)
            y1 = jnp.minimum(y0 + 1, h - 1)
            x1 = jnp.minimum(x0 + 1, w - 1)
            d0 = jnp.abs(tc + 1.0)
            d1 = jnp.abs(tc)
            d2 = jnp.abs(tc - 1.0)
            fi = jnp.where(d1 < d0,
                           jnp.where(d2 < d1, 2, 1),
                           jnp.where(d2 < d0, 2, 0)).astype(jnp.int32)
            za = (fi * d + z0) * hw
            zb = za + (z1 - z0) * hw
            ya = (y0 >> 3) * yts + ((y0 & 7) << 7)
            yb = (y1 >> 3) * yts + ((y1 & 7) << 7)
            xa = ((x0 >> 7) << 10) + (x0 & 127)
            xb = ((x1 >> 7) << 10) + (x1 & 127)
            b00 = za + ya
            b01 = za + yb
            b10 = zb + ya
            b11 = zb + yb
            iref[0][sl] = b00 + xa
            iref[1][sl] = b00 + xb
            iref[2][sl] = b01 + xa
            iref[3][sl] = b01 + xb
            iref[4][sl] = b10 + xa
            iref[5][sl] = b10 + xb
            iref[6][sl] = b11 + xa
            iref[7][sl] = b11 + xb
            return 0

        lax.fori_loop(0, CHUNK // L, idx_body, 0)
        fire_coords(ci + 1, 1 - p)
        for k in range(8):
            pltpu.async_copy(vols_hbm.at[iref[k]], corner_refs[p][k],
                             gsems[p])

    def drain_combine_store(ci, p):
        for k in range(8):
            pltpu.make_async_copy(vols_hbm.at[idx_refs[p][k]],
                                  corner_refs[p][k], gsems[p]).wait()
        cref = corner_refs[p]
        wzr, wyr, wxr = w_refs[p]

        def comb_body(i, _):
            sl = pl.ds(i * L, L)
            wx = wxr[sl]
            wy = wyr[sl]
            wz = wzr[sl]
            c00 = cref[0][sl] * (1 - wx) + cref[1][sl] * wx
            c01 = cref[2][sl] * (1 - wx) + cref[3][sl] * wx
            c10 = cref[4][sl] * (1 - wx) + cref[5][sl] * wx
            c11 = cref[6][sl] * (1 - wx) + cref[7][sl] * wx
            c0 = c00 * (1 - wy) + c01 * wy
            c1 = c10 * (1 - wy) + c11 * wy
            out_v[sl] = c0 * (1 - wz) + c1 * wz
            return 0

        lax.fori_loop(0, CHUNK // L, comb_body, 0)
        cbase = base + ci * CHUNK
        pltpu.sync_copy(out_v, out_hbm.at[pl.ds(cbase, CHUNK)])

    fire_coords(0, 0)
    compute_and_fire(0, 0)

    def pair_body(pi, _):
        ci = pi * 2
        compute_and_fire(ci + 1, 1)
        drain_combine_store(ci, 0)
        compute_and_fire(ci + 2, 0)
        drain_combine_store(ci + 1, 1)
        return 0

    lax.fori_loop(0, n_pairs - 1, pair_body, 0)

    ci = (n_pairs - 1) * 2
    compute_and_fire(ci + 1, 1)
    drain_combine_store(ci, 0)
    drain_combine_store(ci + 1, 1)
    pltpu.make_async_copy(coords_hbm.at[pl.ds(last_cbase * 4, CHUNK * 4)],
                          coord_refs[0], csems[0]).wait()


def kernel(coords, vols):
    n = coords.shape[0]
    nf, d, h, w = vols.shape
    coords_flat = (coords.reshape(n // 128, 128, 4)
                   .transpose(0, 2, 1).reshape(-1))
    vols_flat = (vols.reshape(nf, d, h // 8, 8, w // 128, 128)
                 .transpose(0, 1, 2, 4, 3, 5).reshape(-1))

    mesh = plsc.VectorSubcoreMesh(core_axis_name="c", subcore_axis_name="s",
                                  num_cores=NC, num_subcores=NS)
    body = functools.partial(_interp_kernel, nf, d, h, w, n)
    run = pl.kernel(
        body,
        out_type=jax.ShapeDtypeStruct((n,), jnp.float32),
        mesh=mesh,
        scratch_types=[
            [pltpu.VMEM((4 * CHUNK,), jnp.float32) for _ in range(2)],
            [[pltpu.VMEM((CHUNK,), jnp.int32) for _ in range(8)]
             for _ in range(2)],
            [[pltpu.VMEM((CHUNK,), jnp.float32) for _ in range(8)]
             for _ in range(2)],
            [[pltpu.VMEM((CHUNK,), jnp.float32) for _ in range(3)]
             for _ in range(2)],
            pltpu.VMEM((CHUNK,), jnp.float32),
            pltpu.SemaphoreType.DMA,
            pltpu.SemaphoreType.DMA,
            pltpu.SemaphoreType.DMA,
            pltpu.SemaphoreType.DMA,
        ],
    )
    out = run(coords_flat, vols_flat)
    return out.reshape(n, 1)
